# Initial kernel scaffold; baseline (speedup 1.0000x reference)
import jax
import jax.numpy as jnp
from jax.experimental import pallas as pl


def kernel(x, edge_index, c1w1, c1b1, c1w2, c1b2, c2w1, c2b1, c2w2, c2b2, c3w1, c3b1, c3w2, c3b2, fcw1, fcb1, fcw2, fcb2):
    raise NotImplementedError("write your pallas kernel here")


# speedup vs baseline: 5.0030x; 5.0030x over previous
import functools

import jax
import jax.numpy as jnp
from jax import lax
from jax.experimental import pallas as pl
from jax.experimental.pallas import tpu as pltpu
from jax.experimental.pallas import tpu_sc as plsc

N = 10000
D = 128
E = 320000

_NC = 2
_NS = 16
_NW = _NC * _NS
_EPW = E // _NW
_CHUNK = 80
_NCHUNK = _EPW // _CHUNK
_RPS = 624
_TAIL = N - _NS * _RPS
_TAIL_OFF = _NS * _RPS


def _sc_agg_body(h_hbm, src_hbm, dst_hbm, zero_hbm, out_hbm,
                 idx_s, idx_d, rows, agg_sp, sem):
    cid = lax.axis_index("c")
    sid = lax.axis_index("s")
    wid = sid * _NC + cid

    pltpu.sync_copy(zero_hbm.at[pl.ds(sid * _RPS, _RPS)],
                    agg_sp.at[pl.ds(sid * _RPS, _RPS)])

    @pl.when(sid == 0)
    def _():
        pltpu.sync_copy(zero_hbm.at[pl.ds(_TAIL_OFF, _TAIL)],
                        agg_sp.at[pl.ds(_TAIL_OFF, _TAIL)])

    plsc.subcore_barrier()

    def body(t, carry):
        base = pl.multiple_of(wid * _EPW + t * _CHUNK, _CHUNK)
        pltpu.sync_copy(src_hbm.at[pl.ds(base, _CHUNK)], idx_s)
        pltpu.async_copy(h_hbm.at[idx_s], rows, sem).wait()
        pltpu.sync_copy(dst_hbm.at[pl.ds(base, _CHUNK)], idx_d)
        pltpu.sync_copy(rows, agg_sp.at[idx_d], add=True)
        return carry

    lax.fori_loop(0, _NCHUNK, body, 0)
    plsc.subcore_barrier()
    pltpu.sync_copy(agg_sp.at[pl.ds(sid * _RPS, _RPS)],
                    out_hbm.at[cid, pl.ds(sid * _RPS, _RPS)])

    @pl.when(sid == 0)
    def _():
        pltpu.sync_copy(agg_sp.at[pl.ds(_TAIL_OFF, _TAIL)],
                        out_hbm.at[cid, pl.ds(_TAIL_OFF, _TAIL)])


def _sc_agg(h, src, dst, zeros):
    mesh = plsc.VectorSubcoreMesh(core_axis_name="c", subcore_axis_name="s")
    f = pl.kernel(
        _sc_agg_body,
        mesh=mesh,
        out_type=jax.ShapeDtypeStruct((_NC, N, D), jnp.float32),
        scratch_types=[
            pltpu.VMEM((_CHUNK,), jnp.int32),
            pltpu.VMEM((_CHUNK,), jnp.int32),
            pltpu.VMEM((_CHUNK, D), jnp.float32),
            pltpu.VMEM_SHARED((N, D), jnp.float32),
            pltpu.SemaphoreType.DMA,
        ],
    )
    return f(h, src, dst, zeros)


_BLK = 1000


def _mlp_body(h_ref, p_ref, w1_ref, b1_ref, w2_ref, b2_ref, o_ref):
    hv = h_ref[...] + p_ref[0] + p_ref[1]
    z = jnp.dot(hv, w1_ref[...], preferred_element_type=jnp.float32)
    z = jnp.maximum(z + b1_ref[...], 0.0)
    z = jnp.dot(z, w2_ref[...], preferred_element_type=jnp.float32)
    o_ref[...] = jnp.maximum(z + b2_ref[...], 0.0)


def _tc_mlp(h, p, w1, b1, w2, b2):
    return pl.pallas_call(
        _mlp_body,
        grid=(N // _BLK,),
        in_specs=[
            pl.BlockSpec((_BLK, D), lambda i: (i, 0)),
            pl.BlockSpec((_NC, _BLK, D), lambda i: (0, i, 0)),
            pl.BlockSpec((D, D), lambda i: (0, 0)),
            pl.BlockSpec((1, D), lambda i: (0, 0)),
            pl.BlockSpec((D, D), lambda i: (0, 0)),
            pl.BlockSpec((1, D), lambda i: (0, 0)),
        ],
        out_specs=pl.BlockSpec((_BLK, D), lambda i: (i, 0)),
        out_shape=jax.ShapeDtypeStruct((N, D), jnp.float32),
    )(h, p, w1, b1.reshape(1, D), w2, b2.reshape(1, D))


def _final_body(h_ref, p_ref, w1_ref, b1_ref, w2_ref, b2_ref,
                fw1_ref, fb1_ref, fw2_ref, fb2_ref, o_ref):
    hv = h_ref[...] + p_ref[0] + p_ref[1]
    z = jnp.dot(hv, w1_ref[...], preferred_element_type=jnp.float32)
    z = jnp.maximum(z + b1_ref[...], 0.0)
    z = jnp.dot(z, w2_ref[...], preferred_element_type=jnp.float32)
    z = jnp.maximum(z + b2_ref[...], 0.0)
    z = jnp.dot(z, fw1_ref[...], preferred_element_type=jnp.float32)
    z = jnp.maximum(z + fb1_ref[...], 0.0)
    z = jnp.dot(z, fw2_ref[...], preferred_element_type=jnp.float32)
    o_ref[...] = z + fb2_ref[...]


def _tc_final(h, p, w1, b1, w2, b2, fw1, fb1, fw2, fb2):
    wspec = pl.BlockSpec((D, D), lambda i: (0, 0))
    bspec = pl.BlockSpec((1, D), lambda i: (0, 0))
    return pl.pallas_call(
        _final_body,
        grid=(N // _BLK,),
        in_specs=[
            pl.BlockSpec((_BLK, D), lambda i: (i, 0)),
            pl.BlockSpec((_NC, _BLK, D), lambda i: (0, i, 0)),
            wspec, bspec, wspec, bspec, wspec, bspec, wspec, bspec,
        ],
        out_specs=pl.BlockSpec((_BLK, D), lambda i: (i, 0)),
        out_shape=jax.ShapeDtypeStruct((N, D), jnp.float32),
    )(h, p, w1, b1.reshape(1, D), w2, b2.reshape(1, D),
      fw1, fb1.reshape(1, D), fw2, fb2.reshape(1, D))


def kernel(x, edge_index, c1w1, c1b1, c1w2, c1b2, c2w1, c2b1, c2w2, c2b2,
           c3w1, c3b1, c3w2, c3b2, fcw1, fcb1, fcw2, fcb2):
    src = edge_index[0]
    dst = edge_index[1]
    zeros = jnp.zeros((N, D), jnp.float32)
    p = _sc_agg(x, src, dst, zeros)
    h = _tc_mlp(x, p, c1w1, c1b1, c1w2, c1b2)
    p = _sc_agg(h, src, dst, zeros)
    h = _tc_mlp(h, p, c2w1, c2b1, c2w2, c2b2)
    p = _sc_agg(h, src, dst, zeros)
    return _tc_final(h, p, c3w1, c3b1, c3w2, c3b2, fcw1, fcb1, fcw2, fcb2)


# speedup vs baseline: 11.3078x; 2.2602x over previous
import functools

import jax
import jax.numpy as jnp
from jax import lax
from jax.experimental import pallas as pl
from jax.experimental.pallas import tpu as pltpu
from jax.experimental.pallas import tpu_sc as plsc

N = 10000
D = 128
E = 320000

_NC = 2
_NS = 16
_NW = _NC * _NS
_CHUNK = 128
_NCHUNK = 80
_EPW = _NCHUNK * _CHUNK
_EPAD = _NW * _EPW
_NPADROW = 240
_NP = N + _NPADROW
_RPS = 624
_TAIL = N - _NS * _RPS
_TAIL_OFF = _NS * _RPS


def _sc_agg_body(h_hbm, pk_hbm, zero_hbm, out_hbm,
                 ib0, ib1, rows0, rows1, agg_sp,
                 isem0, isem1, gsem0, gsem1, ssem0, ssem1):
    cid = lax.axis_index("c")
    sid = lax.axis_index("s")
    wid = sid * _NC + cid

    pltpu.async_copy(pk_hbm.at[wid, 0], ib0, isem0).wait()
    pltpu.async_copy(h_hbm.at[ib0.at[0]], rows0, gsem0)
    pltpu.async_copy(pk_hbm.at[wid, 1], ib1, isem1)

    pltpu.sync_copy(zero_hbm.at[pl.ds(sid * _RPS, _RPS)],
                    agg_sp.at[pl.ds(sid * _RPS, _RPS)])

    @pl.when(sid == 0)
    def _():
        pltpu.sync_copy(zero_hbm.at[pl.ds(_TAIL_OFF, _TAIL)],
                        agg_sp.at[pl.ds(_TAIL_OFF, _TAIL)])

    plsc.subcore_barrier()

    def body(k, carry):
        t0 = 2 * k
        t1 = t0 + 1
        pltpu.make_async_copy(pk_hbm.at[wid, t1], ib1, isem1).wait()
        pltpu.async_copy(h_hbm.at[ib1.at[0]], rows1, gsem1)
        pltpu.make_async_copy(h_hbm.at[ib0.at[0]], rows0, gsem0).wait()
        s0 = pltpu.async_copy(rows0, agg_sp.at[ib0.at[1]], ssem0, add=True)
        s0.wait()
        pltpu.async_copy(pk_hbm.at[wid, t0 + 2], ib0, isem0)
        pltpu.make_async_copy(pk_hbm.at[wid, t0 + 2], ib0, isem0).wait()
        pltpu.async_copy(h_hbm.at[ib0.at[0]], rows0, gsem0)
        pltpu.make_async_copy(h_hbm.at[ib1.at[0]], rows1, gsem1).wait()
        s1 = pltpu.async_copy(rows1, agg_sp.at[ib1.at[1]], ssem1, add=True)
        s1.wait()
        pltpu.async_copy(pk_hbm.at[wid, t0 + 3], ib1, isem1)
        return carry

    lax.fori_loop(0, _NCHUNK // 2 - 1, body, 0)

    tl = _NCHUNK - 1
    pltpu.make_async_copy(pk_hbm.at[wid, tl], ib1, isem1).wait()
    pltpu.async_copy(h_hbm.at[ib1.at[0]], rows1, gsem1)
    pltpu.make_async_copy(h_hbm.at[ib0.at[0]], rows0, gsem0).wait()
    pltpu.async_copy(rows0, agg_sp.at[ib0.at[1]], ssem0, add=True).wait()
    pltpu.make_async_copy(h_hbm.at[ib1.at[0]], rows1, gsem1).wait()
    pltpu.async_copy(rows1, agg_sp.at[ib1.at[1]], ssem1, add=True).wait()
    plsc.subcore_barrier()
    pltpu.sync_copy(agg_sp.at[pl.ds(sid * _RPS, _RPS)],
                    out_hbm.at[cid, pl.ds(sid * _RPS, _RPS)])

    @pl.when(sid == 0)
    def _():
        pltpu.sync_copy(agg_sp.at[pl.ds(_TAIL_OFF, _TAIL)],
                        out_hbm.at[cid, pl.ds(_TAIL_OFF, _TAIL)])


def _sc_agg(h, pk, zeros):
    mesh = plsc.VectorSubcoreMesh(core_axis_name="c", subcore_axis_name="s")
    f = pl.kernel(
        _sc_agg_body,
        mesh=mesh,
        out_type=jax.ShapeDtypeStruct((_NC, N, D), jnp.float32),
        scratch_types=[
            pltpu.VMEM((2, _CHUNK), jnp.int32),
            pltpu.VMEM((2, _CHUNK), jnp.int32),
            pltpu.VMEM((_CHUNK, D), jnp.float32),
            pltpu.VMEM((_CHUNK, D), jnp.float32),
            pltpu.VMEM_SHARED((_NP, D), jnp.float32),
            pltpu.SemaphoreType.DMA,
            pltpu.SemaphoreType.DMA,
            pltpu.SemaphoreType.DMA,
            pltpu.SemaphoreType.DMA,
            pltpu.SemaphoreType.DMA,
            pltpu.SemaphoreType.DMA,
        ],
    )
    return f(h, pk, zeros)


_BLK = 1000


def _mlp_body(h_ref, p_ref, w1_ref, b1_ref, w2_ref, b2_ref, o_ref):
    hv = h_ref[...] + p_ref[0] + p_ref[1]
    z = jnp.dot(hv, w1_ref[...], preferred_element_type=jnp.float32)
    z = jnp.maximum(z + b1_ref[...], 0.0)
    z = jnp.dot(z, w2_ref[...], preferred_element_type=jnp.float32)
    o_ref[...] = jnp.maximum(z + b2_ref[...], 0.0)


def _tc_mlp(h, p, w1, b1, w2, b2):
    return pl.pallas_call(
        _mlp_body,
        grid=(N // _BLK,),
        in_specs=[
            pl.BlockSpec((_BLK, D), lambda i: (i, 0)),
            pl.BlockSpec((_NC, _BLK, D), lambda i: (0, i, 0)),
            pl.BlockSpec((D, D), lambda i: (0, 0)),
            pl.BlockSpec((1, D), lambda i: (0, 0)),
            pl.BlockSpec((D, D), lambda i: (0, 0)),
            pl.BlockSpec((1, D), lambda i: (0, 0)),
        ],
        out_specs=pl.BlockSpec((_BLK, D), lambda i: (i, 0)),
        out_shape=jax.ShapeDtypeStruct((N, D), jnp.float32),
    )(h, p, w1, b1.reshape(1, D), w2, b2.reshape(1, D))


def _final_body(h_ref, p_ref, w1_ref, b1_ref, w2_ref, b2_ref,
                fw1_ref, fb1_ref, fw2_ref, fb2_ref, o_ref):
    hv = h_ref[...] + p_ref[0] + p_ref[1]
    z = jnp.dot(hv, w1_ref[...], preferred_element_type=jnp.float32)
    z = jnp.maximum(z + b1_ref[...], 0.0)
    z = jnp.dot(z, w2_ref[...], preferred_element_type=jnp.float32)
    z = jnp.maximum(z + b2_ref[...], 0.0)
    z = jnp.dot(z, fw1_ref[...], preferred_element_type=jnp.float32)
    z = jnp.maximum(z + fb1_ref[...], 0.0)
    z = jnp.dot(z, fw2_ref[...], preferred_element_type=jnp.float32)
    o_ref[...] = z + fb2_ref[...]


def _tc_final(h, p, w1, b1, w2, b2, fw1, fb1, fw2, fb2):
    wspec = pl.BlockSpec((D, D), lambda i: (0, 0))
    bspec = pl.BlockSpec((1, D), lambda i: (0, 0))
    return pl.pallas_call(
        _final_body,
        grid=(N // _BLK,),
        in_specs=[
            pl.BlockSpec((_BLK, D), lambda i: (i, 0)),
            pl.BlockSpec((_NC, _BLK, D), lambda i: (0, i, 0)),
            wspec, bspec, wspec, bspec, wspec, bspec, wspec, bspec,
        ],
        out_specs=pl.BlockSpec((_BLK, D), lambda i: (i, 0)),
        out_shape=jax.ShapeDtypeStruct((N, D), jnp.float32),
    )(h, p, w1, b1.reshape(1, D), w2, b2.reshape(1, D),
      fw1, fb1.reshape(1, D), fw2, fb2.reshape(1, D))


def kernel(x, edge_index, c1w1, c1b1, c1w2, c1b2, c2w1, c2b1, c2w2, c2b2,
           c3w1, c3b1, c3w2, c3b2, fcw1, fcb1, fcw2, fcb2):
    npad = _EPAD - E
    pad_src = jnp.arange(npad, dtype=jnp.int32) % _NPADROW
    pad_dst = N + pad_src
    src = jnp.concatenate([edge_index[0], pad_src]).reshape(_NW, _NCHUNK,
                                                            _CHUNK)
    dst = jnp.concatenate([edge_index[1], pad_dst]).reshape(_NW, _NCHUNK,
                                                            _CHUNK)
    pk = jnp.stack([src, dst], axis=2)
    zeros = jnp.zeros((N, D), jnp.float32)
    p = _sc_agg(x, pk, zeros)
    h = _tc_mlp(x, p, c1w1, c1b1, c1w2, c1b2)
    p = _sc_agg(h, pk, zeros)
    h = _tc_mlp(h, p, c2w1, c2b1, c2w2, c2b2)
    p = _sc_agg(h, pk, zeros)
    return _tc_final(h, p, c3w1, c3b1, c3w2, c3b2, fcw1, fcb1, fcw2, fcb2)


# speedup vs baseline: 13.4238x; 1.1871x over previous
import functools

import jax
import jax.numpy as jnp
from jax import lax
from jax.experimental import pallas as pl
from jax.experimental.pallas import tpu as pltpu
from jax.experimental.pallas import tpu_sc as plsc

N = 10000
D = 128
E = 320000

_NC = 2
_NS = 16
_NW = _NC * _NS
_CHUNK = 96
_NCHUNK = 108
_EPW = _NCHUNK * _CHUNK
_EPAD = _NW * _EPW
_NPADROW = 240
_NP = N + _NPADROW
_RPS = 624
_TAIL = N - _NS * _RPS
_TAIL_OFF = _NS * _RPS


def _sc_agg_body(h_hbm, pk_hbm, zero_hbm, out_hbm,
                 ib0, ib1, ib2, ib3, ib4, ib5, rows0, rows1, rows2, agg_sp,
                 isem0, isem1, isem2, isem3, isem4, isem5,
                 gsem0, gsem1, gsem2, ssem0, ssem1, ssem2):
    cid = lax.axis_index("c")
    sid = lax.axis_index("s")
    wid = sid * _NC + cid

    ibs = [ib0, ib1, ib2, ib3, ib4, ib5]
    isems = [isem0, isem1, isem2, isem3, isem4, isem5]
    rows = [rows0, rows1, rows2]
    gsems = [gsem0, gsem1, gsem2]
    ssems = [ssem0, ssem1, ssem2]

    def fire_i(t, b):
        pltpu.async_copy(pk_hbm.at[wid, t], ibs[b], isems[b])

    def wait_i(t, b):
        pltpu.make_async_copy(pk_hbm.at[wid, t], ibs[b], isems[b]).wait()

    def fire_g(b):
        pltpu.async_copy(h_hbm.at[ibs[b].at[0]], rows[b % 3], gsems[b % 3])

    def wait_g(b):
        pltpu.make_async_copy(h_hbm.at[ibs[b].at[0]], rows[b % 3],
                              gsems[b % 3]).wait()

    def fire_s(b):
        pltpu.async_copy(rows[b % 3], agg_sp.at[ibs[b].at[1]],
                         ssems[b % 3], add=True)

    def wait_s(b):
        pltpu.make_async_copy(rows[b % 3], agg_sp.at[ibs[b].at[1]],
                              ssems[b % 3]).wait()

    def step(t, b, first=False, fi=True, fg=True):
        if not first:
            wait_s((b - 1) % 6)
        if fi:
            fire_i(t + 5, (b + 5) % 6)
        if fg:
            wait_i(t + 2, (b + 2) % 6)
            fire_g((b + 2) % 6)
        wait_g(b)
        fire_s(b)

    for t in range(5):
        fire_i(t, t)
    wait_i(0, 0)
    fire_g(0)
    wait_i(1, 1)
    fire_g(1)

    pltpu.sync_copy(zero_hbm.at[pl.ds(sid * _RPS, _RPS)],
                    agg_sp.at[pl.ds(sid * _RPS, _RPS)])

    @pl.when(sid == 0)
    def _():
        pltpu.sync_copy(zero_hbm.at[pl.ds(_TAIL_OFF, _TAIL)],
                        agg_sp.at[pl.ds(_TAIL_OFF, _TAIL)])

    plsc.subcore_barrier()

    for t in range(6):
        step(t, t, first=(t == 0))

    def body(k, carry):
        t0 = 6 * k
        for j in range(6):
            step(t0 + j, j)
        return carry

    lax.fori_loop(1, _NCHUNK // 6 - 1, body, 0)

    for t in range(_NCHUNK - 6, _NCHUNK):
        step(t, t % 6, fi=(t + 5 < _NCHUNK), fg=(t + 2 < _NCHUNK))
    wait_s((_NCHUNK - 1) % 6)
    plsc.subcore_barrier()
    pltpu.sync_copy(agg_sp.at[pl.ds(sid * _RPS, _RPS)],
                    out_hbm.at[cid, pl.ds(sid * _RPS, _RPS)])

    @pl.when(sid == 0)
    def _():
        pltpu.sync_copy(agg_sp.at[pl.ds(_TAIL_OFF, _TAIL)],
                        out_hbm.at[cid, pl.ds(_TAIL_OFF, _TAIL)])


def _sc_agg(h, pk, zeros):
    mesh = plsc.VectorSubcoreMesh(core_axis_name="c", subcore_axis_name="s")
    f = pl.kernel(
        _sc_agg_body,
        mesh=mesh,
        out_type=jax.ShapeDtypeStruct((_NC, N, D), jnp.float32),
        scratch_types=(
            [pltpu.VMEM((2, _CHUNK), jnp.int32)] * 6
            + [pltpu.VMEM((_CHUNK, D), jnp.float32)] * 3
            + [pltpu.VMEM_SHARED((_NP, D), jnp.float32)]
            + [pltpu.SemaphoreType.DMA] * 12
        ),
    )
    return f(h, pk, zeros)


_BLK = 1000


def _mlp_body(h_ref, p_ref, w1_ref, b1_ref, w2_ref, b2_ref, o_ref):
    hv = h_ref[...] + p_ref[0] + p_ref[1]
    z = jnp.dot(hv, w1_ref[...], preferred_element_type=jnp.float32)
    z = jnp.maximum(z + b1_ref[...], 0.0)
    z = jnp.dot(z, w2_ref[...], preferred_element_type=jnp.float32)
    o_ref[...] = jnp.maximum(z + b2_ref[...], 0.0)


def _tc_mlp(h, p, w1, b1, w2, b2):
    return pl.pallas_call(
        _mlp_body,
        grid=(N // _BLK,),
        in_specs=[
            pl.BlockSpec((_BLK, D), lambda i: (i, 0)),
            pl.BlockSpec((_NC, _BLK, D), lambda i: (0, i, 0)),
            pl.BlockSpec((D, D), lambda i: (0, 0)),
            pl.BlockSpec((1, D), lambda i: (0, 0)),
            pl.BlockSpec((D, D), lambda i: (0, 0)),
            pl.BlockSpec((1, D), lambda i: (0, 0)),
        ],
        out_specs=pl.BlockSpec((_BLK, D), lambda i: (i, 0)),
        out_shape=jax.ShapeDtypeStruct((N, D), jnp.float32),
    )(h, p, w1, b1.reshape(1, D), w2, b2.reshape(1, D))


def _final_body(h_ref, p_ref, w1_ref, b1_ref, w2_ref, b2_ref,
                fw1_ref, fb1_ref, fw2_ref, fb2_ref, o_ref):
    hv = h_ref[...] + p_ref[0] + p_ref[1]
    z = jnp.dot(hv, w1_ref[...], preferred_element_type=jnp.float32)
    z = jnp.maximum(z + b1_ref[...], 0.0)
    z = jnp.dot(z, w2_ref[...], preferred_element_type=jnp.float32)
    z = jnp.maximum(z + b2_ref[...], 0.0)
    z = jnp.dot(z, fw1_ref[...], preferred_element_type=jnp.float32)
    z = jnp.maximum(z + fb1_ref[...], 0.0)
    z = jnp.dot(z, fw2_ref[...], preferred_element_type=jnp.float32)
    o_ref[...] = z + fb2_ref[...]


def _tc_final(h, p, w1, b1, w2, b2, fw1, fb1, fw2, fb2):
    wspec = pl.BlockSpec((D, D), lambda i: (0, 0))
    bspec = pl.BlockSpec((1, D), lambda i: (0, 0))
    return pl.pallas_call(
        _final_body,
        grid=(N // _BLK,),
        in_specs=[
            pl.BlockSpec((_BLK, D), lambda i: (i, 0)),
            pl.BlockSpec((_NC, _BLK, D), lambda i: (0, i, 0)),
            wspec, bspec, wspec, bspec, wspec, bspec, wspec, bspec,
        ],
        out_specs=pl.BlockSpec((_BLK, D), lambda i: (i, 0)),
        out_shape=jax.ShapeDtypeStruct((N, D), jnp.float32),
    )(h, p, w1, b1.reshape(1, D), w2, b2.reshape(1, D),
      fw1, fb1.reshape(1, D), fw2, fb2.reshape(1, D))


def kernel(x, edge_index, c1w1, c1b1, c1w2, c1b2, c2w1, c2b1, c2w2, c2b2,
           c3w1, c3b1, c3w2, c3b2, fcw1, fcb1, fcw2, fcb2):
    npad = _EPAD - E
    pad_idx = jnp.arange(npad, dtype=jnp.int32)
    pad_src = pad_idx % N
    pad_dst = N + pad_idx % _NPADROW
    src = jnp.concatenate([edge_index[0], pad_src]).reshape(_NW, _NCHUNK,
                                                            _CHUNK)
    dst = jnp.concatenate([edge_index[1], pad_dst]).reshape(_NW, _NCHUNK,
                                                            _CHUNK)
    pk = jnp.stack([src, dst], axis=2)
    zeros = jnp.zeros((N, D), jnp.float32)
    p = _sc_agg(x, pk, zeros)
    h = _tc_mlp(x, p, c1w1, c1b1, c1w2, c1b2)
    p = _sc_agg(h, pk, zeros)
    h = _tc_mlp(h, p, c2w1, c2b1, c2w2, c2b2)
    p = _sc_agg(h, pk, zeros)
    return _tc_final(h, p, c3w1, c3b1, c3w2, c3b2, fcw1, fcb1, fcw2, fcb2)


# speedup vs baseline: 14.0741x; 1.0484x over previous
import functools

import jax
import jax.numpy as jnp
from jax import lax
from jax.experimental import pallas as pl
from jax.experimental.pallas import tpu as pltpu
from jax.experimental.pallas import tpu_sc as plsc

N = 10000
D = 128
E = 320000

_NC = 2
_NS = 16
_NW = _NC * _NS
_CHUNK = 80
_NCHUNK = 128
_NROW = 4
_NIB = 8
_EPW = _NCHUNK * _CHUNK
_EPAD = _NW * _EPW
_NPADROW = 240
_NP = N + _NPADROW
_RPS = 624
_TAIL = N - _NS * _RPS
_TAIL_OFF = _NS * _RPS


def _sc_agg_body(h_hbm, pk_hbm, zero_hbm, out_hbm, *sc):
    cid = lax.axis_index("c")
    sid = lax.axis_index("s")
    wid = sid * _NC + cid

    ibs = sc[0:_NIB]
    rows = sc[_NIB:_NIB + _NROW]
    agg_sp = sc[_NIB + _NROW]
    isems = sc[_NIB + _NROW + 1:2 * _NIB + _NROW + 1]
    gsems = sc[2 * _NIB + _NROW + 1:2 * _NIB + 2 * _NROW + 1]
    ssems = sc[2 * _NIB + 2 * _NROW + 1:2 * _NIB + 3 * _NROW + 1]

    _GL = _NROW - 1
    _IL = _NIB - 1

    def fire_i(t, b):
        pltpu.async_copy(pk_hbm.at[wid, t], ibs[b], isems[b])

    def wait_i(t, b):
        pltpu.make_async_copy(pk_hbm.at[wid, t], ibs[b], isems[b]).wait()

    def fire_g(b):
        pltpu.async_copy(h_hbm.at[ibs[b].at[0]], rows[b % _NROW],
                         gsems[b % _NROW])

    def wait_g(b):
        pltpu.make_async_copy(h_hbm.at[ibs[b].at[0]], rows[b % _NROW],
                              gsems[b % _NROW]).wait()

    def fire_s(b):
        pltpu.async_copy(rows[b % _NROW], agg_sp.at[ibs[b].at[1]],
                         ssems[b % _NROW], add=True)

    def wait_s(b):
        pltpu.make_async_copy(rows[b % _NROW], agg_sp.at[ibs[b].at[1]],
                              ssems[b % _NROW]).wait()

    def step(t, b, first=False, fi=True, fg=True):
        if not first:
            wait_s((b - 1) % _NIB)
        if fi:
            fire_i(t + _IL, (b + _IL) % _NIB)
        if fg:
            wait_i(t + _GL, (b + _GL) % _NIB)
            fire_g((b + _GL) % _NIB)
        wait_g(b)
        fire_s(b)

    for t in range(_IL):
        fire_i(t, t)
    for t in range(_GL):
        wait_i(t, t)
        fire_g(t)

    pltpu.sync_copy(zero_hbm.at[pl.ds(sid * _RPS, _RPS)],
                    agg_sp.at[pl.ds(sid * _RPS, _RPS)])

    @pl.when(sid == 0)
    def _():
        pltpu.sync_copy(zero_hbm.at[pl.ds(_TAIL_OFF, _TAIL)],
                        agg_sp.at[pl.ds(_TAIL_OFF, _TAIL)])

    plsc.subcore_barrier()

    for t in range(_NIB):
        step(t, t, first=(t == 0))

    def body(k, carry):
        t0 = _NIB * k
        for j in range(_NIB):
            step(t0 + j, j)
        return carry

    lax.fori_loop(1, _NCHUNK // _NIB - 1, body, 0)

    for t in range(_NCHUNK - _NIB, _NCHUNK):
        step(t, t % _NIB, fi=(t + _IL < _NCHUNK), fg=(t + _GL < _NCHUNK))
    wait_s((_NCHUNK - 1) % _NIB)
    plsc.subcore_barrier()
    pltpu.sync_copy(agg_sp.at[pl.ds(sid * _RPS, _RPS)],
                    out_hbm.at[cid, pl.ds(sid * _RPS, _RPS)])

    @pl.when(sid == 0)
    def _():
        pltpu.sync_copy(agg_sp.at[pl.ds(_TAIL_OFF, _TAIL)],
                        out_hbm.at[cid, pl.ds(_TAIL_OFF, _TAIL)])


def _sc_agg(h, pk, zeros):
    mesh = plsc.VectorSubcoreMesh(core_axis_name="c", subcore_axis_name="s")
    f = pl.kernel(
        _sc_agg_body,
        mesh=mesh,
        out_type=jax.ShapeDtypeStruct((_NC, N, D), jnp.float32),
        scratch_types=(
            [pltpu.VMEM((2, _CHUNK), jnp.int32)] * _NIB
            + [pltpu.VMEM((_CHUNK, D), jnp.float32)] * _NROW
            + [pltpu.VMEM_SHARED((_NP, D), jnp.float32)]
            + [pltpu.SemaphoreType.DMA] * (_NIB + 2 * _NROW)
        ),
    )
    return f(h, pk, zeros)


_BLK = 1000


def _mlp_body(h_ref, p_ref, w1_ref, b1_ref, w2_ref, b2_ref, o_ref):
    hv = h_ref[...] + p_ref[0] + p_ref[1]
    z = jnp.dot(hv, w1_ref[...], preferred_element_type=jnp.float32)
    z = jnp.maximum(z + b1_ref[...], 0.0)
    z = jnp.dot(z, w2_ref[...], preferred_element_type=jnp.float32)
    o_ref[...] = jnp.maximum(z + b2_ref[...], 0.0)


def _tc_mlp(h, p, w1, b1, w2, b2):
    return pl.pallas_call(
        _mlp_body,
        grid=(N // _BLK,),
        in_specs=[
            pl.BlockSpec((_BLK, D), lambda i: (i, 0)),
            pl.BlockSpec((_NC, _BLK, D), lambda i: (0, i, 0)),
            pl.BlockSpec((D, D), lambda i: (0, 0)),
            pl.BlockSpec((1, D), lambda i: (0, 0)),
            pl.BlockSpec((D, D), lambda i: (0, 0)),
            pl.BlockSpec((1, D), lambda i: (0, 0)),
        ],
        out_specs=pl.BlockSpec((_BLK, D), lambda i: (i, 0)),
        out_shape=jax.ShapeDtypeStruct((N, D), jnp.float32),
    )(h, p, w1, b1.reshape(1, D), w2, b2.reshape(1, D))


def _final_body(h_ref, p_ref, w1_ref, b1_ref, w2_ref, b2_ref,
                fw1_ref, fb1_ref, fw2_ref, fb2_ref, o_ref):
    hv = h_ref[...] + p_ref[0] + p_ref[1]
    z = jnp.dot(hv, w1_ref[...], preferred_element_type=jnp.float32)
    z = jnp.maximum(z + b1_ref[...], 0.0)
    z = jnp.dot(z, w2_ref[...], preferred_element_type=jnp.float32)
    z = jnp.maximum(z + b2_ref[...], 0.0)
    z = jnp.dot(z, fw1_ref[...], preferred_element_type=jnp.float32)
    z = jnp.maximum(z + fb1_ref[...], 0.0)
    z = jnp.dot(z, fw2_ref[...], preferred_element_type=jnp.float32)
    o_ref[...] = z + fb2_ref[...]


def _tc_final(h, p, w1, b1, w2, b2, fw1, fb1, fw2, fb2):
    wspec = pl.BlockSpec((D, D), lambda i: (0, 0))
    bspec = pl.BlockSpec((1, D), lambda i: (0, 0))
    return pl.pallas_call(
        _final_body,
        grid=(N // _BLK,),
        in_specs=[
            pl.BlockSpec((_BLK, D), lambda i: (i, 0)),
            pl.BlockSpec((_NC, _BLK, D), lambda i: (0, i, 0)),
            wspec, bspec, wspec, bspec, wspec, bspec, wspec, bspec,
        ],
        out_specs=pl.BlockSpec((_BLK, D), lambda i: (i, 0)),
        out_shape=jax.ShapeDtypeStruct((N, D), jnp.float32),
    )(h, p, w1, b1.reshape(1, D), w2, b2.reshape(1, D),
      fw1, fb1.reshape(1, D), fw2, fb2.reshape(1, D))


def kernel(x, edge_index, c1w1, c1b1, c1w2, c1b2, c2w1, c2b1, c2w2, c2b2,
           c3w1, c3b1, c3w2, c3b2, fcw1, fcb1, fcw2, fcb2):
    npad = _EPAD - E
    pad_idx = jnp.arange(npad, dtype=jnp.int32)
    pad_src = pad_idx % N
    pad_dst = N + pad_idx % _NPADROW
    src = jnp.concatenate([edge_index[0], pad_src]).reshape(_NW, _NCHUNK,
                                                            _CHUNK)
    dst = jnp.concatenate([edge_index[1], pad_dst]).reshape(_NW, _NCHUNK,
                                                            _CHUNK)
    pk = jnp.stack([src, dst], axis=2)
    zeros = jnp.zeros((N, D), jnp.float32)
    p = _sc_agg(x, pk, zeros)
    h = _tc_mlp(x, p, c1w1, c1b1, c1w2, c1b2)
    p = _sc_agg(h, pk, zeros)
    h = _tc_mlp(h, p, c2w1, c2b1, c2w2, c2b2)
    p = _sc_agg(h, pk, zeros)
    return _tc_final(h, p, c3w1, c3b1, c3w2, c3b2, fcw1, fcb1, fcw2, fcb2)


# speedup vs baseline: 14.8327x; 1.0539x over previous
import functools

import jax
import jax.numpy as jnp
from jax import lax
from jax.experimental import pallas as pl
from jax.experimental.pallas import tpu as pltpu
from jax.experimental.pallas import tpu_sc as plsc

N = 10000
D = 128
E = 320000

_NC = 2
_NS = 16
_NW = _NC * _NS
_CHUNK = 112
_NCHUNK = 90
_NROW = 3
_NIB = 6
_EPW = _NCHUNK * _CHUNK
_EPAD = _NW * _EPW
_NPADROW = 128
_NP = N + _NPADROW
_RPS = 624
_TAIL = N - _NS * _RPS
_TAIL_OFF = _NS * _RPS


def _sc_agg_body(h_hbm, ei_hbm, zero_hbm, out_hbm, *sc):
    cid = lax.axis_index("c")
    sid = lax.axis_index("s")
    wid = sid * _NC + cid

    sibs = sc[0:_NIB]
    dibs = sc[_NIB:2 * _NIB]
    rows = sc[2 * _NIB:2 * _NIB + _NROW]
    agg_sp = sc[2 * _NIB + _NROW]
    o = 2 * _NIB + _NROW + 1
    isems_s = sc[o:o + _NIB]
    isems_d = sc[o + _NIB:o + 2 * _NIB]
    gsems = sc[o + 2 * _NIB:o + 2 * _NIB + _NROW]
    ssems = sc[o + 2 * _NIB + _NROW:o + 2 * _NIB + 2 * _NROW]

    _GL = _NROW - 1
    _IL = _NIB - 1

    def fire_i(t, b):
        pltpu.async_copy(ei_hbm.at[0, wid, t], sibs[b], isems_s[b])
        pltpu.async_copy(ei_hbm.at[1, wid, t], dibs[b], isems_d[b])

    def wait_i(t, b):
        pltpu.make_async_copy(ei_hbm.at[0, wid, t], sibs[b],
                              isems_s[b]).wait()
        pltpu.make_async_copy(ei_hbm.at[1, wid, t], dibs[b],
                              isems_d[b]).wait()

    def fire_g(b):
        pltpu.async_copy(h_hbm.at[sibs[b]], rows[b % _NROW],
                         gsems[b % _NROW])

    def wait_g(b):
        pltpu.make_async_copy(h_hbm.at[sibs[b]], rows[b % _NROW],
                              gsems[b % _NROW]).wait()

    def fire_s(b):
        pltpu.async_copy(rows[b % _NROW], agg_sp.at[dibs[b]],
                         ssems[b % _NROW], add=True)

    def wait_s(b):
        pltpu.make_async_copy(rows[b % _NROW], agg_sp.at[dibs[b]],
                              ssems[b % _NROW]).wait()

    def step(t, b, first=False, fi=True, fg=True):
        if not first:
            wait_s((b - 1) % _NIB)
        if fi:
            fire_i(t + _IL, (b + _IL) % _NIB)
        if fg:
            wait_i(t + _GL, (b + _GL) % _NIB)
            fire_g((b + _GL) % _NIB)
        wait_g(b)
        fire_s(b)

    for t in range(_IL):
        fire_i(t, t)
    for t in range(_GL):
        wait_i(t, t)
        fire_g(t)

    pltpu.sync_copy(zero_hbm.at[pl.ds(sid * _RPS, _RPS)],
                    agg_sp.at[pl.ds(sid * _RPS, _RPS)])

    @pl.when(sid == 0)
    def _():
        pltpu.sync_copy(zero_hbm.at[pl.ds(_TAIL_OFF, _TAIL)],
                        agg_sp.at[pl.ds(_TAIL_OFF, _TAIL)])

    plsc.subcore_barrier()

    for t in range(_NIB):
        step(t, t, first=(t == 0))

    def body(k, carry):
        t0 = _NIB * k
        for j in range(_NIB):
            step(t0 + j, j)
        return carry

    lax.fori_loop(1, _NCHUNK // _NIB - 1, body, 0)

    for t in range(_NCHUNK - _NIB, _NCHUNK):
        step(t, t % _NIB, fi=(t + _IL < _NCHUNK), fg=(t + _GL < _NCHUNK))
    wait_s((_NCHUNK - 1) % _NIB)
    plsc.subcore_barrier()
    pltpu.sync_copy(agg_sp.at[pl.ds(sid * _RPS, _RPS)],
                    out_hbm.at[cid, pl.ds(sid * _RPS, _RPS)])

    @pl.when(sid == 0)
    def _():
        pltpu.sync_copy(agg_sp.at[pl.ds(_TAIL_OFF, _TAIL)],
                        out_hbm.at[cid, pl.ds(_TAIL_OFF, _TAIL)])


def _sc_agg(h, ei4, zeros):
    mesh = plsc.VectorSubcoreMesh(core_axis_name="c", subcore_axis_name="s")
    f = pl.kernel(
        _sc_agg_body,
        mesh=mesh,
        out_type=jax.ShapeDtypeStruct((_NC, N, D), jnp.float32),
        scratch_types=(
            [pltpu.VMEM((_CHUNK,), jnp.int32)] * (2 * _NIB)
            + [pltpu.VMEM((_CHUNK, D), jnp.float32)] * _NROW
            + [pltpu.VMEM_SHARED((_NP, D), jnp.float32)]
            + [pltpu.SemaphoreType.DMA] * (2 * _NIB + 2 * _NROW)
        ),
    )
    return f(h, ei4, zeros)


_BLK = 2000


def _mlp_body(h_ref, p_ref, w1_ref, b1_ref, w2_ref, b2_ref, o_ref):
    hv = h_ref[...] + p_ref[0] + p_ref[1]
    z = jnp.dot(hv, w1_ref[...], preferred_element_type=jnp.float32)
    z = jnp.maximum(z + b1_ref[...], 0.0)
    z = jnp.dot(z, w2_ref[...], preferred_element_type=jnp.float32)
    o_ref[...] = jnp.maximum(z + b2_ref[...], 0.0)


def _tc_mlp(h, p, w1, b1, w2, b2):
    return pl.pallas_call(
        _mlp_body,
        grid=(N // _BLK,),
        in_specs=[
            pl.BlockSpec((_BLK, D), lambda i: (i, 0)),
            pl.BlockSpec((_NC, _BLK, D), lambda i: (0, i, 0)),
            pl.BlockSpec((D, D), lambda i: (0, 0)),
            pl.BlockSpec((1, D), lambda i: (0, 0)),
            pl.BlockSpec((D, D), lambda i: (0, 0)),
            pl.BlockSpec((1, D), lambda i: (0, 0)),
        ],
        out_specs=pl.BlockSpec((_BLK, D), lambda i: (i, 0)),
        out_shape=jax.ShapeDtypeStruct((N, D), jnp.float32),
    )(h, p, w1, b1.reshape(1, D), w2, b2.reshape(1, D))


def _final_body(h_ref, p_ref, w1_ref, b1_ref, w2_ref, b2_ref,
                fw1_ref, fb1_ref, fw2_ref, fb2_ref, o_ref):
    hv = h_ref[...] + p_ref[0] + p_ref[1]
    z = jnp.dot(hv, w1_ref[...], preferred_element_type=jnp.float32)
    z = jnp.maximum(z + b1_ref[...], 0.0)
    z = jnp.dot(z, w2_ref[...], preferred_element_type=jnp.float32)
    z = jnp.maximum(z + b2_ref[...], 0.0)
    z = jnp.dot(z, fw1_ref[...], preferred_element_type=jnp.float32)
    z = jnp.maximum(z + fb1_ref[...], 0.0)
    z = jnp.dot(z, fw2_ref[...], preferred_element_type=jnp.float32)
    o_ref[...] = z + fb2_ref[...]


def _tc_final(h, p, w1, b1, w2, b2, fw1, fb1, fw2, fb2):
    wspec = pl.BlockSpec((D, D), lambda i: (0, 0))
    bspec = pl.BlockSpec((1, D), lambda i: (0, 0))
    return pl.pallas_call(
        _final_body,
        grid=(N // _BLK,),
        in_specs=[
            pl.BlockSpec((_BLK, D), lambda i: (i, 0)),
            pl.BlockSpec((_NC, _BLK, D), lambda i: (0, i, 0)),
            wspec, bspec, wspec, bspec, wspec, bspec, wspec, bspec,
        ],
        out_specs=pl.BlockSpec((_BLK, D), lambda i: (i, 0)),
        out_shape=jax.ShapeDtypeStruct((N, D), jnp.float32),
    )(h, p, w1, b1.reshape(1, D), w2, b2.reshape(1, D),
      fw1, fb1.reshape(1, D), fw2, fb2.reshape(1, D))


def kernel(x, edge_index, c1w1, c1b1, c1w2, c1b2, c2w1, c2b1, c2w2, c2b2,
           c3w1, c3b1, c3w2, c3b2, fcw1, fcb1, fcw2, fcb2):
    npad = _EPAD - E
    pad_idx = jnp.arange(npad, dtype=jnp.int32)
    pad = jnp.stack([pad_idx & 8191, N + (pad_idx & (_NPADROW - 1))])
    ei4 = jnp.concatenate([edge_index, pad], axis=1).reshape(
        2, _NW, _NCHUNK, _CHUNK)
    zeros = jnp.zeros((N, D), jnp.float32)
    p = _sc_agg(x, ei4, zeros)
    h = _tc_mlp(x, p, c1w1, c1b1, c1w2, c1b2)
    p = _sc_agg(h, ei4, zeros)
    h = _tc_mlp(h, p, c2w1, c2b1, c2w2, c2b2)
    p = _sc_agg(h, ei4, zeros)
    return _tc_final(h, p, c3w1, c3b1, c3w2, c3b2, fcw1, fcb1, fcw2, fcb2)
